# Initial kernel scaffold; baseline (speedup 1.0000x reference)
import jax
import jax.numpy as jnp
from jax.experimental import pallas as pl


def kernel(x, edge_index, W1, b1, W2, b2):
    raise NotImplementedError("write your pallas kernel here")


# speedup vs baseline: 15.6563x; 15.6563x over previous
import functools

import jax
import jax.numpy as jnp
from jax.experimental import pallas as pl
from jax.experimental.pallas import tpu as pltpu
from jax.experimental.pallas import tpu_sc as plsc

_N = 10000
_E = 320000
_D = 128
_H = 16
_C = 2

_NC = 2
_NS = 16
_NW = _NC * _NS
_L = 16

_ET = _E // _NW
_K = 80
_NCHUNK = _ET // _K
_NPAD = 10240
_RPT = _NPAD // _NS


def _sc_mesh():
    return plsc.VectorSubcoreMesh(core_axis_name="c", subcore_axis_name="s")


_SC_PARAMS = pltpu.CompilerParams(use_tc_tiling_on_sc=False)


def _sc_degree(dst):
    @functools.partial(
        pl.kernel,
        out_type=jax.ShapeDtypeStruct((_NC, _NPAD, _L), jnp.float32),
        mesh=_sc_mesh(),
        compiler_params=_SC_PARAMS,
        scratch_types=[
            pltpu.VMEM((_K,), jnp.int32),
            pltpu.VMEM((_K, _L), jnp.float32),
            pltpu.VMEM((_RPT, _L), jnp.float32),
            pltpu.VMEM_SHARED((_NPAD, _L), jnp.float32),
        ],
    )
    def deg_kernel(dst_hbm, out_hbm, didx, ones, stage, acc):
        c = jax.lax.axis_index("c")
        s = jax.lax.axis_index("s")
        wid = s * _NC + c

        @pl.loop(0, _RPT)
        def _(i):
            stage[i] = jnp.zeros((_L,), jnp.float32)

        @pl.loop(0, _K)
        def _(i):
            ones[i] = jnp.ones((_L,), jnp.float32)

        pltpu.sync_copy(stage, acc.at[pl.ds(s * _RPT, _RPT)])
        plsc.subcore_barrier()

        base = wid * _ET

        @pl.loop(0, _NCHUNK)
        def _(j):
            pltpu.sync_copy(dst_hbm.at[pl.ds(base + j * _K, _K)], didx)
            pltpu.sync_copy(ones, acc.at[didx], add=True)

        plsc.subcore_barrier()
        pltpu.sync_copy(acc.at[pl.ds(s * _RPT, _RPT)], stage)
        pltpu.sync_copy(stage, out_hbm.at[c].at[pl.ds(s * _RPT, _RPT)])

    return deg_kernel(dst)


def _sc_aggregate(table, src, dst):
    @functools.partial(
        pl.kernel,
        out_type=jax.ShapeDtypeStruct((_NC, _NPAD, _L), jnp.float32),
        mesh=_sc_mesh(),
        compiler_params=_SC_PARAMS,
        scratch_types=[
            pltpu.VMEM((_K,), jnp.int32),
            pltpu.VMEM((_K,), jnp.int32),
            pltpu.VMEM((_K, _L), jnp.float32),
            pltpu.VMEM((_RPT, _L), jnp.float32),
            pltpu.VMEM_SHARED((_NPAD, _L), jnp.float32),
            pltpu.SemaphoreType.DMA,
        ],
    )
    def agg_kernel(table_hbm, src_hbm, dst_hbm, out_hbm,
                   sidx, didx, rows, stage, acc, sem):
        c = jax.lax.axis_index("c")
        s = jax.lax.axis_index("s")
        wid = s * _NC + c

        @pl.loop(0, _RPT)
        def _(i):
            stage[i] = jnp.zeros((_L,), jnp.float32)

        pltpu.sync_copy(stage, acc.at[pl.ds(s * _RPT, _RPT)])
        plsc.subcore_barrier()

        base = wid * _ET

        @pl.loop(0, _NCHUNK)
        def _(j):
            pltpu.sync_copy(src_hbm.at[pl.ds(base + j * _K, _K)], sidx)
            pltpu.sync_copy(dst_hbm.at[pl.ds(base + j * _K, _K)], didx)
            pltpu.async_copy(table_hbm.at[sidx], rows, sem).wait()
            pltpu.sync_copy(rows, acc.at[didx], add=True)

        plsc.subcore_barrier()
        pltpu.sync_copy(acc.at[pl.ds(s * _RPT, _RPT)], stage)
        pltpu.sync_copy(stage, out_hbm.at[c].at[pl.ds(s * _RPT, _RPT)])

    return agg_kernel(table, src, dst)


def _tc_xw(x, W1):
    def body(x_ref, w_ref, o_ref):
        o_ref[...] = jnp.dot(x_ref[...], w_ref[...],
                             preferred_element_type=jnp.float32)

    return pl.pallas_call(
        body,
        out_shape=jax.ShapeDtypeStruct((_N, _H), jnp.float32),
    )(x, W1)


def _tc_scale(degp, xw):
    def body(degp_ref, xw_ref, dis_ref, y1_ref):
        deg = degp_ref[0] + degp_ref[1] + 1.0
        dis = jax.lax.rsqrt(deg)
        dis_ref[...] = dis
        y1_ref[...] = dis * xw_ref[...]

    return pl.pallas_call(
        body,
        out_shape=(
            jax.ShapeDtypeStruct((_N, _H), jnp.float32),
            jax.ShapeDtypeStruct((_N, _H), jnp.float32),
        ),
    )(degp, xw)


def _tc_layer1(agg1p, dis, xw, b1):
    def body(aggp_ref, dis_ref, xw_ref, b1_ref, h_ref, y2_ref):
        dis = dis_ref[...]
        agg = aggp_ref[0] + aggp_ref[1]
        pre = dis * agg + dis * dis * xw_ref[...] + b1_ref[...]
        h = jnp.maximum(pre, 0.0)
        h_ref[...] = h
        y2_ref[...] = dis * h

    return pl.pallas_call(
        body,
        out_shape=(
            jax.ShapeDtypeStruct((_N, _H), jnp.float32),
            jax.ShapeDtypeStruct((_N, _H), jnp.float32),
        ),
    )(agg1p, dis, xw, b1.reshape(1, _H))


def _tc_out(agg2p, dis, h, W2, b2):
    def body(aggp_ref, dis_ref, h_ref, w2_ref, b2_ref, o_ref):
        dis = dis_ref[...]
        agg = aggp_ref[0] + aggp_ref[1]
        t = dis * agg + dis * dis * h_ref[...]
        o = jnp.dot(t, w2_ref[...], preferred_element_type=jnp.float32)
        o = o + b2_ref[...]
        m = jnp.max(o, axis=1, keepdims=True)
        lse = m + jnp.log(jnp.sum(jnp.exp(o - m), axis=1, keepdims=True))
        o_ref[...] = o - lse

    return pl.pallas_call(
        body,
        out_shape=jax.ShapeDtypeStruct((_N, _C), jnp.float32),
    )(agg2p, dis, h, W2, b2.reshape(1, _C))


def kernel(x, edge_index, W1, b1, W2, b2):
    src = edge_index[0].astype(jnp.int32)
    dst = edge_index[1].astype(jnp.int32)

    degp = _sc_degree(dst)[:, :_N]
    xw = _tc_xw(x, W1)
    dis, y1 = _tc_scale(degp, xw)
    agg1p = _sc_aggregate(y1, src, dst)[:, :_N]
    h, y2 = _tc_layer1(agg1p, dis, xw, b1)
    agg2p = _sc_aggregate(y2, src, dst)[:, :_N]
    return _tc_out(agg2p, dis, h, W2, b2)


# speedup vs baseline: 44.5223x; 2.8437x over previous
import functools

import jax
import jax.numpy as jnp
from jax.experimental import pallas as pl
from jax.experimental.pallas import tpu as pltpu
from jax.experimental.pallas import tpu_sc as plsc

_N = 10000
_E = 320000
_D = 128
_H = 16
_C = 2

_NC = 2
_NS = 16
_NW = _NC * _NS
_L = 16

_ET = _E // _NW
_K = 80
_NCHUNK = _ET // _K
_NPAD = 10240
_RPT = _NPAD // _NS


def _sc_mesh():
    return plsc.VectorSubcoreMesh(core_axis_name="c", subcore_axis_name="s")


_SC_PARAMS = pltpu.CompilerParams(use_tc_tiling_on_sc=False)


_NBUF = 4


def _sc_degree(dst2d):
    @functools.partial(
        pl.kernel,
        out_type=jax.ShapeDtypeStruct((_NC, _NPAD, _L), jnp.float32),
        mesh=_sc_mesh(),
        compiler_params=_SC_PARAMS,
        scratch_types=[
            pltpu.VMEM((_NCHUNK, _K), jnp.int32),
            pltpu.VMEM((_K, _L), jnp.float32),
            pltpu.VMEM((_RPT, _L), jnp.float32),
            pltpu.VMEM_SHARED((_NPAD, _L), jnp.float32),
        ],
    )
    def deg_kernel(dst_hbm, out_hbm, didx, ones, stage, acc):
        c = jax.lax.axis_index("c")
        s = jax.lax.axis_index("s")
        wid = s * _NC + c

        @pl.loop(0, _RPT)
        def _(i):
            stage[i] = jnp.zeros((_L,), jnp.float32)

        @pl.loop(0, _K)
        def _(i):
            ones[i] = jnp.ones((_L,), jnp.float32)

        pltpu.sync_copy(stage, acc.at[pl.ds(s * _RPT, _RPT)])
        pltpu.sync_copy(dst_hbm.at[pl.ds(wid * _NCHUNK, _NCHUNK)], didx)
        plsc.subcore_barrier()

        @pl.loop(0, _NCHUNK)
        def _(j):
            pltpu.sync_copy(ones, acc.at[didx.at[j]], add=True)

        plsc.subcore_barrier()
        pltpu.sync_copy(acc.at[pl.ds(s * _RPT, _RPT)], stage)
        pltpu.sync_copy(stage, out_hbm.at[c].at[pl.ds(s * _RPT, _RPT)])

    return deg_kernel(dst2d)


def _sc_aggregate(table, src2d, dst2d):
    @functools.partial(
        pl.kernel,
        out_type=jax.ShapeDtypeStruct((_NC, _NPAD, _L), jnp.float32),
        mesh=_sc_mesh(),
        compiler_params=_SC_PARAMS,
        scratch_types=[
            pltpu.VMEM((_NCHUNK, _K), jnp.int32),
            pltpu.VMEM((_NCHUNK, _K), jnp.int32),
            pltpu.VMEM((_NBUF, _K, _L), jnp.float32),
            pltpu.VMEM((_RPT, _L), jnp.float32),
            pltpu.VMEM_SHARED((_NPAD, _L), jnp.float32),
        ] + [pltpu.SemaphoreType.DMA] * _NBUF,
    )
    def agg_kernel(table_hbm, src_hbm, dst_hbm, out_hbm,
                   sidx, didx, rows, stage, acc, *gsems):
        c = jax.lax.axis_index("c")
        s = jax.lax.axis_index("s")
        wid = s * _NC + c

        @pl.loop(0, _RPT)
        def _(i):
            stage[i] = jnp.zeros((_L,), jnp.float32)

        pltpu.sync_copy(src_hbm.at[pl.ds(wid * _NCHUNK, _NCHUNK)], sidx)
        pltpu.sync_copy(dst_hbm.at[pl.ds(wid * _NCHUNK, _NCHUNK)], didx)
        pltpu.sync_copy(stage, acc.at[pl.ds(s * _RPT, _RPT)])
        plsc.subcore_barrier()

        for b in range(_NBUF):
            pltpu.async_copy(table_hbm.at[sidx.at[b]], rows.at[b], gsems[b])

        @pl.loop(0, _NCHUNK - (_NCHUNK % _NBUF), step=_NBUF)
        def _(g):
            for b in range(_NBUF):
                jj = g + b
                pltpu.make_async_copy(
                    table_hbm.at[sidx.at[jj]], rows.at[b], gsems[b]).wait()
                pltpu.sync_copy(rows.at[b], acc.at[didx.at[jj]], add=True)
                nxt = jj + _NBUF

                @pl.when(nxt < _NCHUNK)
                def _():
                    pltpu.async_copy(
                        table_hbm.at[sidx.at[nxt]], rows.at[b], gsems[b])

        for b in range(_NCHUNK % _NBUF):
            jj = _NCHUNK - (_NCHUNK % _NBUF) + b
            pltpu.make_async_copy(
                table_hbm.at[sidx.at[jj]], rows.at[b], gsems[b]).wait()
            pltpu.sync_copy(rows.at[b], acc.at[didx.at[jj]], add=True)

        plsc.subcore_barrier()
        pltpu.sync_copy(acc.at[pl.ds(s * _RPT, _RPT)], stage)
        pltpu.sync_copy(stage, out_hbm.at[c].at[pl.ds(s * _RPT, _RPT)])

    return agg_kernel(table, src2d, dst2d)


def _tc_xw(x, W1):
    def body(x_ref, w_ref, o_ref):
        o_ref[...] = jnp.dot(x_ref[...], w_ref[...],
                             preferred_element_type=jnp.float32)

    return pl.pallas_call(
        body,
        out_shape=jax.ShapeDtypeStruct((_N, _H), jnp.float32),
    )(x, W1)


def _tc_scale(degp, xw):
    def body(degp_ref, xw_ref, dis_ref, y1_ref):
        deg = degp_ref[0] + degp_ref[1] + 1.0
        dis = jax.lax.rsqrt(deg)
        dis_ref[...] = dis
        y1_ref[...] = dis * xw_ref[...]

    return pl.pallas_call(
        body,
        out_shape=(
            jax.ShapeDtypeStruct((_N, _H), jnp.float32),
            jax.ShapeDtypeStruct((_N, _H), jnp.float32),
        ),
    )(degp, xw)


def _tc_layer1(agg1p, dis, xw, b1):
    def body(aggp_ref, dis_ref, xw_ref, b1_ref, h_ref, y2_ref):
        dis = dis_ref[...]
        agg = aggp_ref[0] + aggp_ref[1]
        pre = dis * agg + dis * dis * xw_ref[...] + b1_ref[...]
        h = jnp.maximum(pre, 0.0)
        h_ref[...] = h
        y2_ref[...] = dis * h

    return pl.pallas_call(
        body,
        out_shape=(
            jax.ShapeDtypeStruct((_N, _H), jnp.float32),
            jax.ShapeDtypeStruct((_N, _H), jnp.float32),
        ),
    )(agg1p, dis, xw, b1.reshape(1, _H))


def _tc_out(agg2p, dis, h, W2, b2):
    def body(aggp_ref, dis_ref, h_ref, w2_ref, b2_ref, o_ref):
        dis = dis_ref[...]
        agg = aggp_ref[0] + aggp_ref[1]
        t = dis * agg + dis * dis * h_ref[...]
        o = jnp.dot(t, w2_ref[...], preferred_element_type=jnp.float32)
        o = o + b2_ref[...]
        m = jnp.max(o, axis=1, keepdims=True)
        lse = m + jnp.log(jnp.sum(jnp.exp(o - m), axis=1, keepdims=True))
        o_ref[...] = o - lse

    return pl.pallas_call(
        body,
        out_shape=jax.ShapeDtypeStruct((_N, _C), jnp.float32),
    )(agg2p, dis, h, W2, b2.reshape(1, _C))


def kernel(x, edge_index, W1, b1, W2, b2):
    src2d = edge_index[0].astype(jnp.int32).reshape(_NW * _NCHUNK, _K)
    dst2d = edge_index[1].astype(jnp.int32).reshape(_NW * _NCHUNK, _K)

    degp = _sc_degree(dst2d)[:, :_N]
    xw = _tc_xw(x, W1)
    dis, y1 = _tc_scale(degp, xw)
    agg1p = _sc_aggregate(y1, src2d, dst2d)[:, :_N]
    h, y2 = _tc_layer1(agg1p, dis, xw, b1)
    agg2p = _sc_aggregate(y2, src2d, dst2d)[:, :_N]
    return _tc_out(agg2p, dis, h, W2, b2)
